# baseline (speedup 1.0000x reference)
import functools

import jax
import jax.numpy as jnp
from jax.experimental import pallas as pl
from jax.experimental.pallas import tpu as pltpu

B, T, D = 16, 4096, 128
TB = 512
NEG_INF = float("-inf")


def _body(lens_ref, x_ref, wt_ref, b_ref, out_ref):
    i = pl.program_id(0)
    j = pl.program_id(1)
    length = lens_ref[i]
    t0 = j * TB

    @pl.when(t0 + TB <= length)
    def _full_valid():
        out_ref[0] = (
            jnp.dot(x_ref[0], wt_ref[...], preferred_element_type=jnp.float32)
            + b_ref[...]
        )

    @pl.when(t0 >= length)
    def _full_pad():
        out_ref[0] = jnp.full((TB, D), NEG_INF, dtype=jnp.float32)

    @pl.when(jnp.logical_and(t0 < length, t0 + TB > length))
    def _partial():
        rows = t0 + jax.lax.broadcasted_iota(jnp.int32, (TB, D), 0)
        y = (
            jnp.dot(x_ref[0], wt_ref[...], preferred_element_type=jnp.float32)
            + b_ref[...]
        )
        out_ref[0] = jnp.where(rows < length, y, NEG_INF)


def _x_index(i, j, lens):
    last = jnp.maximum(pl.cdiv(lens[i], TB) - 1, 0)
    return (i, jnp.minimum(j, last), 0)


@functools.partial(jax.jit, static_argnames=())
def _run(padded, lengths, wt, b2):
    grid_spec = pltpu.PrefetchScalarGridSpec(
        num_scalar_prefetch=1,
        grid=(B, T // TB),
        in_specs=[
            pl.BlockSpec((1, TB, D), _x_index),
            pl.BlockSpec((D, D), lambda i, j, lens: (0, 0)),
            pl.BlockSpec((1, D), lambda i, j, lens: (0, 0)),
        ],
        out_specs=pl.BlockSpec((1, TB, D), lambda i, j, lens: (i, j, 0)),
    )
    out = pl.pallas_call(
        _body,
        grid_spec=grid_spec,
        out_shape=jax.ShapeDtypeStruct((B, T, D), jnp.float32),
    )(lengths, padded, wt, b2)
    return out


def kernel(padded, lengths, W, b):
    wt = W.T
    b2 = b.reshape(1, D)
    out = _run(padded, lengths.astype(jnp.int32), wt, b2)
    return out, lengths


# speedup vs baseline: 1.4315x; 1.4315x over previous
import functools

import jax
import jax.numpy as jnp
from jax.experimental import pallas as pl
from jax.experimental.pallas import tpu as pltpu

B, T, D = 16, 4096, 128
TB = 1024
NEG_INF = float("-inf")


def _body(lens_ref, x_ref, wt_ref, b_ref, out_ref):
    i = pl.program_id(0)
    j = pl.program_id(1)
    length = lens_ref[i]
    t0 = j * TB

    @pl.when(t0 + TB <= length)
    def _full_valid():
        out_ref[0] = (
            jnp.dot(x_ref[0], wt_ref[...], preferred_element_type=jnp.float32)
            + b_ref[...]
        )

    @pl.when(t0 >= length)
    def _full_pad():
        out_ref[0] = jnp.full((TB, D), NEG_INF, dtype=jnp.float32)

    @pl.when(jnp.logical_and(t0 < length, t0 + TB > length))
    def _partial():
        rows = t0 + jax.lax.broadcasted_iota(jnp.int32, (TB, D), 0)
        y = (
            jnp.dot(x_ref[0], wt_ref[...], preferred_element_type=jnp.float32)
            + b_ref[...]
        )
        out_ref[0] = jnp.where(rows < length, y, NEG_INF)


def _x_index(i, j, lens):
    last = jnp.maximum(pl.cdiv(lens[i], TB) - 1, 0)
    return (i, jnp.minimum(j, last), 0)


@functools.partial(jax.jit, static_argnames=())
def _run(padded, lengths, wt, b2):
    grid_spec = pltpu.PrefetchScalarGridSpec(
        num_scalar_prefetch=1,
        grid=(B, T // TB),
        in_specs=[
            pl.BlockSpec((1, TB, D), _x_index),
            pl.BlockSpec((D, D), lambda i, j, lens: (0, 0)),
            pl.BlockSpec((1, D), lambda i, j, lens: (0, 0)),
        ],
        out_specs=pl.BlockSpec((1, TB, D), lambda i, j, lens: (i, j, 0)),
    )
    out = pl.pallas_call(
        _body,
        grid_spec=grid_spec,
        out_shape=jax.ShapeDtypeStruct((B, T, D), jnp.float32),
        compiler_params=pltpu.CompilerParams(
            dimension_semantics=("parallel", "arbitrary"),
        ),
    )(lengths, padded, wt, b2)
    return out


def kernel(padded, lengths, W, b):
    wt = W.T
    b2 = b.reshape(1, D)
    out = _run(padded, lengths.astype(jnp.int32), wt, b2)
    return out, lengths


# speedup vs baseline: 1.8635x; 1.3018x over previous
import functools

import jax
import jax.numpy as jnp
from jax.experimental import pallas as pl
from jax.experimental.pallas import tpu as pltpu

B, T, D = 16, 4096, 128
TB = 2048
NEG_INF = float("-inf")


def _body(lens_ref, x_ref, wt_ref, b_ref, out_ref):
    i = pl.program_id(0)
    j = pl.program_id(1)
    length = lens_ref[i]
    t0 = j * TB

    @pl.when(t0 + TB <= length)
    def _full_valid():
        out_ref[0] = (
            jnp.dot(x_ref[0], wt_ref[...], preferred_element_type=jnp.float32)
            + b_ref[...]
        )

    @pl.when(t0 >= length)
    def _full_pad():
        out_ref[0] = jnp.full((TB, D), NEG_INF, dtype=jnp.float32)

    @pl.when(jnp.logical_and(t0 < length, t0 + TB > length))
    def _partial():
        rows = t0 + jax.lax.broadcasted_iota(jnp.int32, (TB, D), 0)
        y = (
            jnp.dot(x_ref[0], wt_ref[...], preferred_element_type=jnp.float32)
            + b_ref[...]
        )
        out_ref[0] = jnp.where(rows < length, y, NEG_INF)


def _x_index(i, j, lens):
    last = jnp.maximum(pl.cdiv(lens[i], TB) - 1, 0)
    return (i, jnp.minimum(j, last), 0)


@functools.partial(jax.jit, static_argnames=())
def _run(padded, lengths, wt, b2):
    grid_spec = pltpu.PrefetchScalarGridSpec(
        num_scalar_prefetch=1,
        grid=(B, T // TB),
        in_specs=[
            pl.BlockSpec((1, TB, D), _x_index),
            pl.BlockSpec((D, D), lambda i, j, lens: (0, 0)),
            pl.BlockSpec((1, D), lambda i, j, lens: (0, 0)),
        ],
        out_specs=pl.BlockSpec((1, TB, D), lambda i, j, lens: (i, j, 0)),
    )
    out = pl.pallas_call(
        _body,
        grid_spec=grid_spec,
        out_shape=jax.ShapeDtypeStruct((B, T, D), jnp.float32),
        compiler_params=pltpu.CompilerParams(
            dimension_semantics=("parallel", "arbitrary"),
        ),
    )(lengths, padded, wt, b2)
    return out


def kernel(padded, lengths, W, b):
    wt = W.T
    b2 = b.reshape(1, D)
    out = _run(padded, lengths.astype(jnp.int32), wt, b2)
    return out, lengths


# speedup vs baseline: 2.6206x; 1.4063x over previous
import functools

import jax
import jax.numpy as jnp
from jax.experimental import pallas as pl
from jax.experimental.pallas import tpu as pltpu

B, T, D = 16, 4096, 128
TB = 4096
NEG_INF = float("-inf")


def _body(lens_ref, x_ref, wt_ref, b_ref, out_ref):
    i = pl.program_id(0)
    j = pl.program_id(1)
    length = lens_ref[i]
    t0 = j * TB

    @pl.when(t0 + TB <= length)
    def _full_valid():
        out_ref[0] = (
            jnp.dot(x_ref[0], wt_ref[...], preferred_element_type=jnp.float32)
            + b_ref[...]
        )

    @pl.when(t0 >= length)
    def _full_pad():
        out_ref[0] = jnp.full((TB, D), NEG_INF, dtype=jnp.float32)

    @pl.when(jnp.logical_and(t0 < length, t0 + TB > length))
    def _partial():
        rows = t0 + jax.lax.broadcasted_iota(jnp.int32, (TB, D), 0)
        y = (
            jnp.dot(x_ref[0], wt_ref[...], preferred_element_type=jnp.float32)
            + b_ref[...]
        )
        out_ref[0] = jnp.where(rows < length, y, NEG_INF)


def _x_index(i, j, lens):
    last = jnp.maximum(pl.cdiv(lens[i], TB) - 1, 0)
    return (i, jnp.minimum(j, last), 0)


@functools.partial(jax.jit, static_argnames=())
def _run(padded, lengths, wt, b2):
    grid_spec = pltpu.PrefetchScalarGridSpec(
        num_scalar_prefetch=1,
        grid=(B, T // TB),
        in_specs=[
            pl.BlockSpec((1, TB, D), _x_index),
            pl.BlockSpec((D, D), lambda i, j, lens: (0, 0)),
            pl.BlockSpec((1, D), lambda i, j, lens: (0, 0)),
        ],
        out_specs=pl.BlockSpec((1, TB, D), lambda i, j, lens: (i, j, 0)),
    )
    out = pl.pallas_call(
        _body,
        grid_spec=grid_spec,
        out_shape=jax.ShapeDtypeStruct((B, T, D), jnp.float32),
        compiler_params=pltpu.CompilerParams(
            dimension_semantics=("parallel", "arbitrary"),
        ),
    )(lengths, padded, wt, b2)
    return out


def kernel(padded, lengths, W, b):
    wt = W.T
    b2 = b.reshape(1, D)
    out = _run(padded, lengths.astype(jnp.int32), wt, b2)
    return out, lengths
